# baseline (speedup 1.0000x reference)
import functools

import jax
import jax.numpy as jnp
from jax.experimental import pallas as pl
from jax.experimental.pallas import tpu as pltpu

N = 1024
E = 2048
_HI = jax.lax.Precision.HIGHEST


def _wd_body(h_ref, w2_ref, b2_ref, out_ref, *, ib):
    hb = h_ref[...].astype(jnp.bfloat16)
    w2b = w2_ref[...].astype(jnp.bfloat16)
    for t in range(ib):
        acc = jax.lax.dot_general(
            hb, w2b[:, t, :], (((1,), (0,)), ((), ())),
            preferred_element_type=jnp.float32)
        out_ref[:, t, :] = (acc + b2_ref[0, t, :][None, :]).astype(jnp.bfloat16)


def _wd_bf16(h, W2, b2, in_ch, out_ch, ib, eb):
    k = W2.shape[0]
    w2r = W2.reshape(k, in_ch, out_ch)
    b2r = b2.reshape(1, in_ch, out_ch)
    return pl.pallas_call(
        functools.partial(_wd_body, ib=ib),
        grid=(in_ch // ib, E // eb),
        in_specs=[
            pl.BlockSpec((eb, k), lambda i, e: (e, 0)),
            pl.BlockSpec((k, ib, out_ch), lambda i, e: (0, i, 0)),
            pl.BlockSpec((1, ib, out_ch), lambda i, e: (0, i, 0)),
        ],
        out_specs=pl.BlockSpec((eb, ib, out_ch), lambda i, e: (e, i, 0)),
        out_shape=jax.ShapeDtypeStruct((E, in_ch, out_ch), jnp.bfloat16),
        compiler_params=pltpu.CompilerParams(
            dimension_semantics=("arbitrary", "arbitrary")),
    )(h, w2r, b2r)


def _gather_body(src_ref, x_ref, xj_ref, oh):
    iota = jax.lax.broadcasted_iota(jnp.int32, (E, N), 1)
    oh[...] = (iota == src_ref[...]).astype(jnp.float32)
    xj = jax.lax.dot_general(oh[...], x_ref[...], (((1,), (0,)), ((), ())),
                             preferred_element_type=jnp.float32,
                             precision=_HI)
    xj_ref[...] = xj.astype(jnp.bfloat16)


def _gather_bf16(x, src_col):
    cin = x.shape[1]
    return pl.pallas_call(
        _gather_body,
        in_specs=[pl.BlockSpec((E, 1), lambda: (0, 0)),
                  pl.BlockSpec((N, cin), lambda: (0, 0))],
        out_specs=pl.BlockSpec((E, cin), lambda: (0, 0)),
        out_shape=jax.ShapeDtypeStruct((E, cin), jnp.bfloat16),
        scratch_shapes=[pltpu.VMEM((E, N), jnp.float32)],
    )(src_col, x)


def _counts_body(dst_ref, c_ref, oh):
    iota = jax.lax.broadcasted_iota(jnp.int32, (E, N), 1)
    oh[...] = (iota == dst_ref[...]).astype(jnp.float32)
    ones = jnp.ones((1, E), jnp.float32)
    c_ref[...] = jax.lax.dot_general(
        ones, oh[...], (((1,), (0,)), ((), ())),
        preferred_element_type=jnp.float32, precision=_HI)


def _counts(dst_col):
    return pl.pallas_call(
        _counts_body,
        in_specs=[pl.BlockSpec((E, 1), lambda: (0, 0))],
        out_specs=pl.BlockSpec((1, N), lambda: (0, 0)),
        out_shape=jax.ShapeDtypeStruct((1, N), jnp.float32),
        scratch_shapes=[pltpu.VMEM((E, N), jnp.float32)],
    )(dst_col).reshape(N)


def _nnconv(x, src_col, dst, cnt, edge_attr, W1, b1, W2, b2, root, bias,
            in_ch, out_ch, ib, eb):
    h = jax.nn.leaky_relu(jnp.dot(edge_attr, W1) + b1, negative_slope=0.01)
    Wd = _wd_bf16(h, W2, b2, in_ch, out_ch, ib, eb)
    x_j = _gather_bf16(x, src_col)
    msg = jnp.einsum('ei,eio->eo', x_j, Wd,
                     preferred_element_type=jnp.float32)
    s = jax.ops.segment_sum(msg, dst, num_segments=x.shape[0])
    mean = s / jnp.maximum(cnt, 1.0)[:, None]
    return mean + jnp.dot(x, root) + bias


def kernel(x, edge_index, edge_attr, epoch,
           nn1_W1, nn1_b1, nn1_W2, nn1_b2, root1, bias1,
           nn2_W1, nn2_b1, nn2_W2, nn2_b2, root2, bias2,
           nn3_W1, nn3_b1, nn3_W2, nn3_b2, root3, bias3):
    tau = 500.0 / (epoch + 1)
    src_col = edge_index[0].reshape(E, 1)
    dst = edge_index[1]
    cnt = _counts(edge_index[1].reshape(E, 1))
    d1 = jax.nn.leaky_relu(_nnconv(x, src_col, dst, cnt, edge_attr, nn1_W1, nn1_b1, nn1_W2, nn1_b2, root1, bias1, 64, 512, ib=8, eb=512), 0.01)
    d2 = jax.nn.leaky_relu(_nnconv(d1, src_col, dst, cnt, edge_attr, nn2_W1, nn2_b1, nn2_W2, nn2_b2, root2, bias2, 512, 256, ib=16, eb=512), 0.01)
    d3 = jax.nn.leaky_relu(_nnconv(d2, src_col, dst, cnt, edge_attr, nn3_W1, nn3_b1, nn3_W2, nn3_b2, root3, bias3, 256, 64, ib=16, eb=512), 0.01)
    g = jax.random.gumbel(jax.random.key(42), d3.shape, dtype=d3.dtype)
    y_soft = jax.nn.softmax((d3 + g) / tau, axis=-1)
    y_hard = jax.nn.one_hot(jnp.argmax(y_soft, axis=-1), d3.shape[-1], dtype=d3.dtype)
    return y_hard - jax.lax.stop_gradient(y_soft) + y_soft


# speedup vs baseline: 2.3586x; 2.3586x over previous
import functools

import jax
import jax.numpy as jnp
from jax.experimental import pallas as pl
from jax.experimental.pallas import tpu as pltpu

N = 1024
E = 2048
_HI = jax.lax.Precision.HIGHEST


def _wd_body(h_ref, w2_ref, b2_ref, out_ref, *, ib, out_ch):
    hb = h_ref[...].astype(jnp.bfloat16)
    w2b = w2_ref[...].astype(jnp.bfloat16)
    acc = jax.lax.dot_general(
        hb, w2b, (((1,), (0,)), ((), ())),
        preferred_element_type=jnp.float32) + b2_ref[...]
    eb = acc.shape[0]
    out_ref[...] = acc.astype(jnp.bfloat16).reshape(eb, ib, out_ch)


def _wd_bf16(h, W2, b2, in_ch, out_ch, ib, eb):
    k = W2.shape[0]
    b2r = b2.reshape(1, in_ch * out_ch)
    return pl.pallas_call(
        functools.partial(_wd_body, ib=ib, out_ch=out_ch),
        grid=(in_ch // ib, E // eb),
        in_specs=[
            pl.BlockSpec((eb, k), lambda i, e: (e, 0)),
            pl.BlockSpec((k, ib * out_ch), lambda i, e: (0, i)),
            pl.BlockSpec((1, ib * out_ch), lambda i, e: (0, i)),
        ],
        out_specs=pl.BlockSpec((eb, ib, out_ch), lambda i, e: (e, i, 0)),
        out_shape=jax.ShapeDtypeStruct((E, in_ch, out_ch), jnp.bfloat16),
        compiler_params=pltpu.CompilerParams(
            dimension_semantics=("arbitrary", "arbitrary")),
    )(h, W2, b2r)


def _gather_body(src_ref, x_ref, xj_ref, oh):
    iota = jax.lax.broadcasted_iota(jnp.int32, (E, N), 1)
    oh[...] = (iota == src_ref[...]).astype(jnp.float32)
    xj = jax.lax.dot_general(oh[...], x_ref[...], (((1,), (0,)), ((), ())),
                             preferred_element_type=jnp.float32,
                             precision=_HI)
    xj_ref[...] = xj.astype(jnp.bfloat16)


def _gather_bf16(x, src_col):
    cin = x.shape[1]
    return pl.pallas_call(
        _gather_body,
        in_specs=[pl.BlockSpec((E, 1), lambda: (0, 0)),
                  pl.BlockSpec((N, cin), lambda: (0, 0))],
        out_specs=pl.BlockSpec((E, cin), lambda: (0, 0)),
        out_shape=jax.ShapeDtypeStruct((E, cin), jnp.bfloat16),
        scratch_shapes=[pltpu.VMEM((E, N), jnp.float32)],
    )(src_col, x)


def _counts_body(dst_ref, c_ref, oh):
    iota = jax.lax.broadcasted_iota(jnp.int32, (E, N), 1)
    oh[...] = (iota == dst_ref[...]).astype(jnp.float32)
    ones = jnp.ones((1, E), jnp.float32)
    c_ref[...] = jax.lax.dot_general(
        ones, oh[...], (((1,), (0,)), ((), ())),
        preferred_element_type=jnp.float32, precision=_HI)


def _counts(dst_col):
    return pl.pallas_call(
        _counts_body,
        in_specs=[pl.BlockSpec((E, 1), lambda: (0, 0))],
        out_specs=pl.BlockSpec((1, N), lambda: (0, 0)),
        out_shape=jax.ShapeDtypeStruct((1, N), jnp.float32),
        scratch_shapes=[pltpu.VMEM((E, N), jnp.float32)],
    )(dst_col).reshape(N)


def _nnconv(x, src_col, dst, cnt, edge_attr, W1, b1, W2, b2, root, bias,
            in_ch, out_ch, ib, eb):
    h = jax.nn.leaky_relu(jnp.dot(edge_attr, W1) + b1, negative_slope=0.01)
    Wd = _wd_bf16(h, W2, b2, in_ch, out_ch, ib, eb)
    x_j = _gather_bf16(x, src_col)
    msg = jnp.einsum('ei,eio->eo', x_j, Wd,
                     preferred_element_type=jnp.float32)
    s = jax.ops.segment_sum(msg, dst, num_segments=x.shape[0])
    mean = s / jnp.maximum(cnt, 1.0)[:, None]
    return mean + jnp.dot(x, root) + bias


def kernel(x, edge_index, edge_attr, epoch,
           nn1_W1, nn1_b1, nn1_W2, nn1_b2, root1, bias1,
           nn2_W1, nn2_b1, nn2_W2, nn2_b2, root2, bias2,
           nn3_W1, nn3_b1, nn3_W2, nn3_b2, root3, bias3):
    tau = 500.0 / (epoch + 1)
    src_col = edge_index[0].reshape(E, 1)
    dst = edge_index[1]
    cnt = _counts(edge_index[1].reshape(E, 1))
    d1 = jax.nn.leaky_relu(_nnconv(x, src_col, dst, cnt, edge_attr, nn1_W1, nn1_b1, nn1_W2, nn1_b2, root1, bias1, 64, 512, ib=8, eb=512), 0.01)
    d2 = jax.nn.leaky_relu(_nnconv(d1, src_col, dst, cnt, edge_attr, nn2_W1, nn2_b1, nn2_W2, nn2_b2, root2, bias2, 512, 256, ib=16, eb=512), 0.01)
    d3 = jax.nn.leaky_relu(_nnconv(d2, src_col, dst, cnt, edge_attr, nn3_W1, nn3_b1, nn3_W2, nn3_b2, root3, bias3, 256, 64, ib=16, eb=512), 0.01)
    g = jax.random.gumbel(jax.random.key(42), d3.shape, dtype=d3.dtype)
    y_soft = jax.nn.softmax((d3 + g) / tau, axis=-1)
    y_hard = jax.nn.one_hot(jnp.argmax(y_soft, axis=-1), d3.shape[-1], dtype=d3.dtype)
    return y_hard - jax.lax.stop_gradient(y_soft) + y_soft


# speedup vs baseline: 2.3593x; 1.0003x over previous
import functools

import jax
import jax.numpy as jnp
from jax.experimental import pallas as pl
from jax.experimental.pallas import tpu as pltpu

N = 1024
E = 2048
_HI = jax.lax.Precision.HIGHEST


def _wd_body(h_ref, w2_ref, b2_ref, out_ref, *, ib, out_ch):
    hb = h_ref[...].astype(jnp.bfloat16)
    w2b = w2_ref[...].astype(jnp.bfloat16)
    acc = jax.lax.dot_general(
        hb, w2b, (((1,), (0,)), ((), ())),
        preferred_element_type=jnp.float32) + b2_ref[...]
    eb = acc.shape[0]
    out_ref[...] = acc.astype(jnp.bfloat16).reshape(eb, ib, out_ch)


def _wd_bf16(h, W2, b2, in_ch, out_ch, ib, eb):
    k = W2.shape[0]
    b2r = b2.reshape(1, in_ch * out_ch)
    return pl.pallas_call(
        functools.partial(_wd_body, ib=ib, out_ch=out_ch),
        grid=(in_ch // ib, E // eb),
        in_specs=[
            pl.BlockSpec((eb, k), lambda i, e: (e, 0)),
            pl.BlockSpec((k, ib * out_ch), lambda i, e: (0, i)),
            pl.BlockSpec((1, ib * out_ch), lambda i, e: (0, i)),
        ],
        out_specs=pl.BlockSpec((eb, ib, out_ch), lambda i, e: (e, i, 0)),
        out_shape=jax.ShapeDtypeStruct((E, in_ch, out_ch), jnp.bfloat16),
        compiler_params=pltpu.CompilerParams(
            dimension_semantics=("arbitrary", "arbitrary")),
    )(h, W2, b2r)


def _gather_body(src_ref, x_ref, xj_ref, oh):
    iota = jax.lax.broadcasted_iota(jnp.int32, (E, N), 1)
    oh[...] = (iota == src_ref[...]).astype(jnp.float32)
    xj = jax.lax.dot_general(oh[...], x_ref[...], (((1,), (0,)), ((), ())),
                             preferred_element_type=jnp.float32,
                             precision=_HI)
    xj_ref[...] = xj.astype(jnp.bfloat16)


def _gather_bf16(x, src_col):
    cin = x.shape[1]
    return pl.pallas_call(
        _gather_body,
        in_specs=[pl.BlockSpec((E, 1), lambda: (0, 0)),
                  pl.BlockSpec((N, cin), lambda: (0, 0))],
        out_specs=pl.BlockSpec((E, cin), lambda: (0, 0)),
        out_shape=jax.ShapeDtypeStruct((E, cin), jnp.bfloat16),
        scratch_shapes=[pltpu.VMEM((E, N), jnp.float32)],
    )(src_col, x)


def _counts_body(dst_ref, c_ref, oh):
    iota = jax.lax.broadcasted_iota(jnp.int32, (E, N), 1)
    oh[...] = (iota == dst_ref[...]).astype(jnp.float32)
    ones = jnp.ones((1, E), jnp.float32)
    c_ref[...] = jax.lax.dot_general(
        ones, oh[...], (((1,), (0,)), ((), ())),
        preferred_element_type=jnp.float32, precision=_HI)


def _counts(dst_col):
    return pl.pallas_call(
        _counts_body,
        in_specs=[pl.BlockSpec((E, 1), lambda: (0, 0))],
        out_specs=pl.BlockSpec((1, N), lambda: (0, 0)),
        out_shape=jax.ShapeDtypeStruct((1, N), jnp.float32),
        scratch_shapes=[pltpu.VMEM((E, N), jnp.float32)],
    )(dst_col).reshape(N)


def kernel(x, edge_index, edge_attr, epoch,
           nn1_W1, nn1_b1, nn1_W2, nn1_b2, root1, bias1,
           nn2_W1, nn2_b1, nn2_W2, nn2_b2, root2, bias2,
           nn3_W1, nn3_b1, nn3_W2, nn3_b2, root3, bias3):
    tau = 500.0 / (epoch + 1)
    src_col = edge_index[0].reshape(E, 1)
    dst = edge_index[1]
    cnt = _counts(edge_index[1].reshape(E, 1))
    cdiv = jnp.maximum(cnt, 1.0)[:, None]
    h1 = jax.nn.leaky_relu(jnp.dot(edge_attr, nn1_W1) + nn1_b1, negative_slope=0.01)
    h2 = jax.nn.leaky_relu(jnp.dot(edge_attr, nn2_W1) + nn2_b1, negative_slope=0.01)
    h3 = jax.nn.leaky_relu(jnp.dot(edge_attr, nn3_W1) + nn3_b1, negative_slope=0.01)

    Wd1 = _wd_bf16(h1, nn1_W2, nn1_b2, 64, 512, ib=8, eb=512)
    msg1 = jnp.einsum('ei,eio->eo', _gather_bf16(x, src_col), Wd1,
                      preferred_element_type=jnp.float32)
    Wd2 = _wd_bf16(h2, nn2_W2, nn2_b2, 512, 256, ib=16, eb=512)
    s1 = jax.ops.segment_sum(msg1, dst, num_segments=N)
    d1 = jax.nn.leaky_relu(s1 / cdiv + jnp.dot(x, root1) + bias1, 0.01)

    msg2 = jnp.einsum('ei,eio->eo', _gather_bf16(d1, src_col), Wd2,
                      preferred_element_type=jnp.float32)
    Wd3 = _wd_bf16(h3, nn3_W2, nn3_b2, 256, 64, ib=16, eb=512)
    s2 = jax.ops.segment_sum(msg2, dst, num_segments=N)
    d2 = jax.nn.leaky_relu(s2 / cdiv + jnp.dot(d1, root2) + bias2, 0.01)

    msg3 = jnp.einsum('ei,eio->eo', _gather_bf16(d2, src_col), Wd3,
                      preferred_element_type=jnp.float32)
    s3 = jax.ops.segment_sum(msg3, dst, num_segments=N)
    d3 = jax.nn.leaky_relu(s3 / cdiv + jnp.dot(d2, root3) + bias3, 0.01)
    g = jax.random.gumbel(jax.random.key(42), d3.shape, dtype=d3.dtype)
    y_soft = jax.nn.softmax((d3 + g) / tau, axis=-1)
    y_hard = jax.nn.one_hot(jnp.argmax(y_soft, axis=-1), d3.shape[-1], dtype=d3.dtype)
    return y_hard - jax.lax.stop_gradient(y_soft) + y_soft


# speedup vs baseline: 2.6697x; 1.1316x over previous
import functools

import jax
import jax.numpy as jnp
from jax.experimental import pallas as pl
from jax.experimental.pallas import tpu as pltpu

N = 1024
E = 2048
_HI = jax.lax.Precision.HIGHEST


def _wd_body(h_ref, w2_ref, b2_ref, out_ref, *, ib, out_ch):
    hb = h_ref[...]
    w2b = w2_ref[...].astype(jnp.bfloat16)
    acc = jax.lax.dot_general(
        hb, w2b, (((1,), (0,)), ((), ())),
        preferred_element_type=jnp.float32) + b2_ref[...]
    eb = acc.shape[0]
    out_ref[...] = acc.astype(jnp.bfloat16).reshape(eb, ib, out_ch)


def _wd_bf16(h, W2, b2, in_ch, out_ch, ib, eb):
    k = W2.shape[0]
    b2r = b2.reshape(1, in_ch * out_ch)
    return pl.pallas_call(
        functools.partial(_wd_body, ib=ib, out_ch=out_ch),
        grid=(in_ch // ib, E // eb),
        in_specs=[
            pl.BlockSpec((eb, k), lambda i, e: (e, 0)),
            pl.BlockSpec((k, ib * out_ch), lambda i, e: (0, i)),
            pl.BlockSpec((1, ib * out_ch), lambda i, e: (0, i)),
        ],
        out_specs=pl.BlockSpec((eb, ib, out_ch), lambda i, e: (e, i, 0)),
        out_shape=jax.ShapeDtypeStruct((E, in_ch, out_ch), jnp.bfloat16),
        compiler_params=pltpu.CompilerParams(
            dimension_semantics=("arbitrary", "arbitrary")),
    )(h.astype(jnp.bfloat16), W2, b2r)


def _gather_body(src_ref, x_ref, xj_ref, oh):
    iota = jax.lax.broadcasted_iota(jnp.int32, (E, N), 1)
    oh[...] = (iota == src_ref[...]).astype(jnp.float32)
    xj = jax.lax.dot_general(oh[...], x_ref[...], (((1,), (0,)), ((), ())),
                             preferred_element_type=jnp.float32,
                             precision=_HI)
    xj_ref[...] = xj.astype(jnp.bfloat16)


def _gather_bf16(x, src_col):
    cin = x.shape[1]
    return pl.pallas_call(
        _gather_body,
        in_specs=[pl.BlockSpec((E, 1), lambda: (0, 0)),
                  pl.BlockSpec((N, cin), lambda: (0, 0))],
        out_specs=pl.BlockSpec((E, cin), lambda: (0, 0)),
        out_shape=jax.ShapeDtypeStruct((E, cin), jnp.bfloat16),
        scratch_shapes=[pltpu.VMEM((E, N), jnp.float32)],
    )(src_col, x)


def _counts_body(dst_ref, c_ref, oh):
    iota = jax.lax.broadcasted_iota(jnp.int32, (E, N), 1)
    oh[...] = (iota == dst_ref[...]).astype(jnp.float32)
    ones = jnp.ones((1, E), jnp.float32)
    c_ref[...] = jax.lax.dot_general(
        ones, oh[...], (((1,), (0,)), ((), ())),
        preferred_element_type=jnp.float32, precision=_HI)


def _counts(dst_col):
    return pl.pallas_call(
        _counts_body,
        in_specs=[pl.BlockSpec((E, 1), lambda: (0, 0))],
        out_specs=pl.BlockSpec((1, N), lambda: (0, 0)),
        out_shape=jax.ShapeDtypeStruct((1, N), jnp.float32),
        scratch_shapes=[pltpu.VMEM((E, N), jnp.float32)],
    )(dst_col).reshape(N)


def kernel(x, edge_index, edge_attr, epoch,
           nn1_W1, nn1_b1, nn1_W2, nn1_b2, root1, bias1,
           nn2_W1, nn2_b1, nn2_W2, nn2_b2, root2, bias2,
           nn3_W1, nn3_b1, nn3_W2, nn3_b2, root3, bias3):
    tau = 500.0 / (epoch + 1)
    src_col = edge_index[0].reshape(E, 1)
    dst = edge_index[1]
    cnt = _counts(edge_index[1].reshape(E, 1))
    cdiv = jnp.maximum(cnt, 1.0)[:, None]
    h1 = jax.nn.leaky_relu(jnp.dot(edge_attr, nn1_W1) + nn1_b1, negative_slope=0.01)
    h2 = jax.nn.leaky_relu(jnp.dot(edge_attr, nn2_W1) + nn2_b1, negative_slope=0.01)
    h3 = jax.nn.leaky_relu(jnp.dot(edge_attr, nn3_W1) + nn3_b1, negative_slope=0.01)

    Wd1 = _wd_bf16(h1, nn1_W2, nn1_b2, 64, 512, ib=8, eb=1024)
    msg1 = jnp.einsum('ei,eio->eo', _gather_bf16(x, src_col), Wd1,
                      preferred_element_type=jnp.float32)
    Wd2 = _wd_bf16(h2, nn2_W2, nn2_b2, 512, 256, ib=16, eb=1024)
    s1 = jax.ops.segment_sum(msg1, dst, num_segments=N)
    d1 = jax.nn.leaky_relu(s1 / cdiv + jnp.dot(x, root1) + bias1, 0.01)

    msg2 = jnp.einsum('ei,eio->eo', _gather_bf16(d1, src_col), Wd2,
                      preferred_element_type=jnp.float32)
    Wd3 = _wd_bf16(h3, nn3_W2, nn3_b2, 256, 64, ib=32, eb=2048)
    s2 = jax.ops.segment_sum(msg2, dst, num_segments=N)
    d2 = jax.nn.leaky_relu(s2 / cdiv + jnp.dot(d1, root2) + bias2, 0.01)

    msg3 = jnp.einsum('ei,eio->eo', _gather_bf16(d2, src_col), Wd3,
                      preferred_element_type=jnp.float32)
    s3 = jax.ops.segment_sum(msg3, dst, num_segments=N)
    d3 = jax.nn.leaky_relu(s3 / cdiv + jnp.dot(d2, root3) + bias3, 0.01)
    g = jax.random.gumbel(jax.random.key(42), d3.shape, dtype=d3.dtype)
    y_soft = jax.nn.softmax((d3 + g) / tau, axis=-1)
    y_hard = jax.nn.one_hot(jnp.argmax(y_soft, axis=-1), d3.shape[-1], dtype=d3.dtype)
    return y_hard - jax.lax.stop_gradient(y_soft) + y_soft
